# baseline (speedup 1.0000x reference)
import functools

import jax
import jax.numpy as jnp
from jax import lax
from jax.experimental import pallas as pl
from jax.experimental.pallas import tpu as pltpu
from jax.experimental.pallas import tpu_sc as plsc

VOCAB = 1000000
EMBED = 64
BATCH = 16384
P = 4
N = 20
ROWS = P + N

NUM_WORKERS = 32
PER_W = BATCH // NUM_WORKERS
CHUNK = 32
NCHUNK = PER_W // CHUNK
CTX = CHUNK * ROWS
GSLICE = 128

NBLK = VOCAB // 128
VTAIL = NBLK * 128
BLK_ITERS = -(-NBLK // NUM_WORKERS)

_MESH = dict(core_axis_name="c", subcore_axis_name="s",
             num_cores=2, num_subcores=16)


def _sc_linearize(ct, bt, ctail, btail):
    @functools.partial(
        pl.kernel,
        mesh=plsc.VectorSubcoreMesh(**_MESH),
        out_type=(jax.ShapeDtypeStruct((VOCAB * EMBED,), jnp.float32),
                  jax.ShapeDtypeStruct((VOCAB * EMBED,), jnp.float32)),
        compiler_params=pltpu.CompilerParams(
            needs_layout_passes=False, use_tc_tiling_on_sc=True),
        scratch_types=(
            [pltpu.VMEM((EMBED, 128), jnp.float32)] * 4
            + [pltpu.VMEM((128 * EMBED,), jnp.float32)] * 4
            + [pltpu.VMEM((EMBED * EMBED,), jnp.float32)]
            + [pltpu.SemaphoreType.DMA] * 8
        ),
    )
    def kl(ct_hbm, bt_hbm, ctail_hbm, btail_hbm, lc_hbm, lb_hbm,
           t0, t1, t2, t3, r0, r1, r2, r3, tail_v,
           si0, si1, si2, si3, so0, so1, so2, so3):
        tiles = [t0, t1, t2, t3]
        rows = [r0, r1, r2, r3]
        si = [si0, si1, si2, si3]
        so = [so0, so1, so2, so3]
        wid = lax.axis_index("s") * 2 + lax.axis_index("c")
        lane = lax.iota(jnp.int32, 16)
        rq = [(16 * q + lane) for q in range(4)]
        nb = jnp.int32(NBLK - 1)

        def blk_at(it):
            return jnp.minimum(wid + NUM_WORKERS * it, nb)

        aq = [(16 * q + lane) for q in range(4)]

        def transpose(tiles_v, rows_v):
            @pl.loop(0, 128, unroll=16)
            def _col(c):
                t = (lane + c) & 127
                u = t * EMBED
                gs = [plsc.load_gather(tiles_v, [rq[q], t]) for q in range(4)]
                for q in range(4):
                    plsc.store_scatter(rows_v, [u + aq[q]], gs[q])

        NITER = -(-(BLK_ITERS) // 4) * 4

        for (tin, tout) in ((ct_hbm, lc_hbm), (bt_hbm, lb_hbm)):
            for u in range(4):
                pltpu.async_copy(tin.at[:, pl.ds(128 * blk_at(u), 128)],
                                 tiles[u], si[u])

            @pl.loop(0, NITER, step=4)
            def _blk(it):
                for u in range(4):
                    b = blk_at(it + u)
                    pltpu.make_async_copy(tin.at[:, pl.ds(0, 128)],
                                          tiles[u], si[u]).wait()

                    @pl.when(it > 0)
                    def _():
                        pltpu.make_async_copy(
                            tout.at[pl.ds(0, 128 * EMBED)], rows[u],
                            so[u]).wait()
                    transpose(tiles[u], rows[u])
                    pltpu.async_copy(
                        rows[u], tout.at[pl.ds(b * 128 * EMBED, 128 * EMBED)],
                        so[u])
                    pltpu.async_copy(
                        tin.at[:, pl.ds(128 * blk_at(it + 4 + u), 128)],
                        tiles[u], si[u])

            for u in range(4):
                pltpu.make_async_copy(tin.at[:, pl.ds(0, 128)],
                                      tiles[u], si[u]).wait()
                pltpu.make_async_copy(tout.at[pl.ds(0, 128 * EMBED)],
                                      rows[u], so[u]).wait()

        @pl.when(wid == 0)
        def _():
            pltpu.sync_copy(ctail_hbm, tail_v)
            pltpu.sync_copy(tail_v,
                            lc_hbm.at[pl.ds(VTAIL * EMBED, EMBED * EMBED)])

        @pl.when(wid == 1)
        def _():
            pltpu.sync_copy(btail_hbm, tail_v)
            pltpu.sync_copy(tail_v,
                            lb_hbm.at[pl.ds(VTAIL * EMBED, EMBED * EMBED)])

    return kl(ct, bt, ctail, btail)


def _sc_dots(input_labels, pos_labels, neg_labels, center, back):
    @functools.partial(
        pl.kernel,
        mesh=plsc.VectorSubcoreMesh(**_MESH),
        out_type=jax.ShapeDtypeStruct((BATCH, 32), jnp.float32),
        compiler_params=pltpu.CompilerParams(
            needs_layout_passes=False, use_tc_tiling_on_sc=False),
        scratch_types=[
            pltpu.VMEM((CHUNK,), jnp.int32),
            pltpu.VMEM((P, CHUNK), jnp.int32),
            pltpu.VMEM((N, CHUNK), jnp.int32),
            pltpu.VMEM((CTX,), jnp.int32),
            pltpu.VMEM((CHUNK, EMBED), jnp.float32),
            pltpu.VMEM((CTX, EMBED), jnp.float32),
            pltpu.VMEM((CHUNK, 32), jnp.float32),
            pltpu.SemaphoreType.DMA,
            pltpu.SemaphoreType.DMA,
        ],
    )
    def k(inlab_hbm, pos_hbm, neg_hbm, center_hbm, back_hbm, out_hbm,
          idxc_v, idxp_v, idxn_v, idxf_v, in_rows, ctx_rows, dots_v,
          sem_c, sem_b):
        wid = lax.axis_index("s") * 2 + lax.axis_index("c")
        lane = lax.iota(jnp.int32, 16)

        @pl.loop(0, NCHUNK)
        def _chunk(kk):
            base = wid * PER_W + kk * CHUNK
            pltpu.sync_copy(inlab_hbm.at[pl.ds(base, CHUNK)], idxc_v)
            pltpu.sync_copy(pos_hbm.at[:, pl.ds(base, CHUNK)], idxp_v)
            pltpu.sync_copy(neg_hbm.at[:, pl.ds(base, CHUNK)], idxn_v)
            cdesc = pltpu.async_copy(center_hbm.at[idxc_v], in_rows, sem_c)
            for j in range(P):
                for h in range(CHUNK // 16):
                    idxf_v[pl.ds(j * CHUNK + 16 * h, 16)] = (
                        idxp_v[j, pl.ds(16 * h, 16)])
            for j in range(N):
                for h in range(CHUNK // 16):
                    idxf_v[pl.ds(CHUNK * P + j * CHUNK + 16 * h, 16)] = (
                        idxn_v[j, pl.ds(16 * h, 16)])
            gds = []
            for j in range(CTX // GSLICE):
                gds.append(pltpu.async_copy(
                    back_hbm.at[idxf_v.at[pl.ds(j * GSLICE, GSLICE)]],
                    ctx_rows.at[pl.ds(j * GSLICE, GSLICE)],
                    sem_b))
            cdesc.wait()
            for d in gds:
                d.wait()

            @pl.loop(0, CHUNK)
            def _elem(b):
                ins = [in_rows[b, pl.ds(16 * q, 16)] for q in range(4)]
                lo = jnp.zeros((16,), jnp.float32)
                hi = jnp.zeros((16,), jnp.float32)
                for r in range(ROWS):
                    if r < P:
                        ro = r * CHUNK + b
                    else:
                        ro = CHUNK * P + (r - P) * CHUNK + b
                    acc = ins[0] * ctx_rows[ro, pl.ds(0, 16)]
                    for q in range(1, 4):
                        acc = acc + ins[q] * ctx_rows[ro, pl.ds(16 * q, 16)]
                    d = jnp.sum(acc)
                    if r < 16:
                        lo = jnp.where(lane == r, d, lo)
                    else:
                        hi = jnp.where(lane == (r - 16), d, hi)
                dots_v[b, pl.ds(0, 16)] = lo
                dots_v[b, pl.ds(16, 16)] = hi

            pltpu.sync_copy(dots_v, out_hbm.at[pl.ds(base, CHUNK)])

    return k(input_labels, pos_labels, neg_labels, center, back)


def _logsig(x):
    return jnp.minimum(x, 0.0) - jnp.log1p(jnp.exp(-jnp.abs(x)))


def _loss_body(d_ref, o_ref):
    x = d_ref[...]
    pos = x[:, 0:P]
    neg = x[:, P:ROWS]
    lp = jnp.sum(_logsig(pos), axis=1)
    ln = jnp.sum(_logsig(-neg), axis=1)
    o_ref[...] = -(lp + ln)


def _tc_loss(dots):
    blk = 1024
    return pl.pallas_call(
        _loss_body,
        grid=(BATCH // blk,),
        in_specs=[pl.BlockSpec((blk, 32), lambda i: (i, 0))],
        out_specs=pl.BlockSpec((blk,), lambda i: (i,)),
        out_shape=jax.ShapeDtypeStruct((BATCH,), jnp.float32),
    )(dots)


def kernel(input_labels, pos_labels, neg_labels, center_embedding, back_embedding):
    ctail = center_embedding[VTAIL:].reshape(-1)
    btail = back_embedding[VTAIL:].reshape(-1)
    lc, lb = _sc_linearize(center_embedding.T, back_embedding.T, ctail, btail)
    dots = _sc_dots(input_labels.astype(jnp.int32),
                    pos_labels.astype(jnp.int32).T,
                    neg_labels.astype(jnp.int32).T,
                    lc.reshape(VOCAB, EMBED),
                    lb.reshape(VOCAB, EMBED))
    return _tc_loss(dots)


# speedup vs baseline: 1.0895x; 1.0895x over previous
import functools

import jax
import jax.numpy as jnp
from jax import lax
from jax.experimental import pallas as pl
from jax.experimental.pallas import tpu as pltpu
from jax.experimental.pallas import tpu_sc as plsc

VOCAB = 1000000
EMBED = 64
BATCH = 16384
P = 4
N = 20
ROWS = P + N

NUM_WORKERS = 32
PER_W = BATCH // NUM_WORKERS
CHUNK = 32
NCHUNK = PER_W // CHUNK
CTX = CHUNK * ROWS
GSLICE = 128

NBLK = VOCAB // 128
VTAIL = NBLK * 128
BLK_ITERS = -(-NBLK // NUM_WORKERS)

_MESH = dict(core_axis_name="c", subcore_axis_name="s",
             num_cores=2, num_subcores=16)


def _sc_linearize(ct, bt, ctail, btail):
    @functools.partial(
        pl.kernel,
        mesh=plsc.VectorSubcoreMesh(**_MESH),
        out_type=(jax.ShapeDtypeStruct((VOCAB * EMBED,), jnp.float32),
                  jax.ShapeDtypeStruct((VOCAB * EMBED,), jnp.float32)),
        compiler_params=pltpu.CompilerParams(
            needs_layout_passes=False, use_tc_tiling_on_sc=True),
        scratch_types=(
            [pltpu.VMEM((EMBED, 128), jnp.float32)] * 4
            + [pltpu.VMEM((128 * EMBED,), jnp.float32)] * 4
            + [pltpu.VMEM((EMBED * EMBED,), jnp.float32)]
            + [pltpu.SemaphoreType.DMA] * 8
        ),
    )
    def kl(ct_hbm, bt_hbm, ctail_hbm, btail_hbm, lc_hbm, lb_hbm,
           t0, t1, t2, t3, r0, r1, r2, r3, tail_v,
           si0, si1, si2, si3, so0, so1, so2, so3):
        tiles = [t0, t1, t2, t3]
        rows = [r0, r1, r2, r3]
        si = [si0, si1, si2, si3]
        so = [so0, so1, so2, so3]
        wid = lax.axis_index("s") * 2 + lax.axis_index("c")
        lane = lax.iota(jnp.int32, 16)
        rq = [(16 * q + lane) for q in range(4)]
        nb = jnp.int32(NBLK - 1)

        def blk_at(it):
            return jnp.minimum(wid + NUM_WORKERS * it, nb)

        aq = [(16 * q + lane) for q in range(4)]

        def transpose(tiles_v, rows_v):
            @pl.loop(0, 128, unroll=8)
            def _col(c):
                t = (lane + c) & 127
                u = t * EMBED
                gs = [plsc.load_gather(tiles_v, [rq[q], t]) for q in range(4)]
                for q in range(4):
                    plsc.store_scatter(rows_v, [u + aq[q]], gs[q])

        NITER = -(-(BLK_ITERS) // 4) * 4

        for (tin, tout) in ((ct_hbm, lc_hbm), (bt_hbm, lb_hbm)):
            for u in range(4):
                pltpu.async_copy(tin.at[:, pl.ds(128 * blk_at(u), 128)],
                                 tiles[u], si[u])

            @pl.loop(0, NITER, step=4)
            def _blk(it):
                for u in range(4):
                    b = blk_at(it + u)
                    pltpu.make_async_copy(tin.at[:, pl.ds(0, 128)],
                                          tiles[u], si[u]).wait()

                    @pl.when(it > 0)
                    def _():
                        pltpu.make_async_copy(
                            tout.at[pl.ds(0, 128 * EMBED)], rows[u],
                            so[u]).wait()
                    transpose(tiles[u], rows[u])
                    pltpu.async_copy(
                        rows[u], tout.at[pl.ds(b * 128 * EMBED, 128 * EMBED)],
                        so[u])
                    pltpu.async_copy(
                        tin.at[:, pl.ds(128 * blk_at(it + 4 + u), 128)],
                        tiles[u], si[u])

            for u in range(4):
                pltpu.make_async_copy(tin.at[:, pl.ds(0, 128)],
                                      tiles[u], si[u]).wait()
                pltpu.make_async_copy(tout.at[pl.ds(0, 128 * EMBED)],
                                      rows[u], so[u]).wait()

        @pl.when(wid == 0)
        def _():
            pltpu.sync_copy(ctail_hbm, tail_v)
            pltpu.sync_copy(tail_v,
                            lc_hbm.at[pl.ds(VTAIL * EMBED, EMBED * EMBED)])

        @pl.when(wid == 1)
        def _():
            pltpu.sync_copy(btail_hbm, tail_v)
            pltpu.sync_copy(tail_v,
                            lb_hbm.at[pl.ds(VTAIL * EMBED, EMBED * EMBED)])

    return kl(ct, bt, ctail, btail)


def _sc_dots(input_labels, pos_labels, neg_labels, center, back):
    @functools.partial(
        pl.kernel,
        mesh=plsc.VectorSubcoreMesh(**_MESH),
        out_type=jax.ShapeDtypeStruct((BATCH, 32), jnp.float32),
        compiler_params=pltpu.CompilerParams(
            needs_layout_passes=False, use_tc_tiling_on_sc=False),
        scratch_types=[
            pltpu.VMEM((CHUNK,), jnp.int32),
            pltpu.VMEM((P, CHUNK), jnp.int32),
            pltpu.VMEM((N, CHUNK), jnp.int32),
            pltpu.VMEM((CTX,), jnp.int32),
            pltpu.VMEM((CHUNK, EMBED), jnp.float32),
            pltpu.VMEM((CTX, EMBED), jnp.float32),
            pltpu.VMEM((CHUNK, 32), jnp.float32),
            pltpu.SemaphoreType.DMA,
            pltpu.SemaphoreType.DMA,
        ],
    )
    def k(inlab_hbm, pos_hbm, neg_hbm, center_hbm, back_hbm, out_hbm,
          idxc_v, idxp_v, idxn_v, idxf_v, in_rows, ctx_rows, dots_v,
          sem_c, sem_b):
        wid = lax.axis_index("s") * 2 + lax.axis_index("c")
        lane = lax.iota(jnp.int32, 16)

        @pl.loop(0, NCHUNK)
        def _chunk(kk):
            base = wid * PER_W + kk * CHUNK
            pltpu.sync_copy(inlab_hbm.at[pl.ds(base, CHUNK)], idxc_v)
            pltpu.sync_copy(pos_hbm.at[:, pl.ds(base, CHUNK)], idxp_v)
            pltpu.sync_copy(neg_hbm.at[:, pl.ds(base, CHUNK)], idxn_v)
            cdesc = pltpu.async_copy(center_hbm.at[idxc_v], in_rows, sem_c)
            for j in range(P):
                for h in range(CHUNK // 16):
                    idxf_v[pl.ds(j * CHUNK + 16 * h, 16)] = (
                        idxp_v[j, pl.ds(16 * h, 16)])
            for j in range(N):
                for h in range(CHUNK // 16):
                    idxf_v[pl.ds(CHUNK * P + j * CHUNK + 16 * h, 16)] = (
                        idxn_v[j, pl.ds(16 * h, 16)])
            gds = []
            for j in range(CTX // GSLICE):
                gds.append(pltpu.async_copy(
                    back_hbm.at[idxf_v.at[pl.ds(j * GSLICE, GSLICE)]],
                    ctx_rows.at[pl.ds(j * GSLICE, GSLICE)],
                    sem_b))
            cdesc.wait()
            for d in gds:
                d.wait()

            @pl.loop(0, CHUNK)
            def _elem(b):
                ins = [in_rows[b, pl.ds(16 * q, 16)] for q in range(4)]
                lo = jnp.zeros((16,), jnp.float32)
                hi = jnp.zeros((16,), jnp.float32)
                for r in range(ROWS):
                    if r < P:
                        ro = r * CHUNK + b
                    else:
                        ro = CHUNK * P + (r - P) * CHUNK + b
                    acc = ins[0] * ctx_rows[ro, pl.ds(0, 16)]
                    for q in range(1, 4):
                        acc = acc + ins[q] * ctx_rows[ro, pl.ds(16 * q, 16)]
                    d = jnp.sum(acc)
                    if r < 16:
                        lo = jnp.where(lane == r, d, lo)
                    else:
                        hi = jnp.where(lane == (r - 16), d, hi)
                dots_v[b, pl.ds(0, 16)] = lo
                dots_v[b, pl.ds(16, 16)] = hi

            pltpu.sync_copy(dots_v, out_hbm.at[pl.ds(base, CHUNK)])

    return k(input_labels, pos_labels, neg_labels, center, back)


def _logsig(x):
    return jnp.minimum(x, 0.0) - jnp.log1p(jnp.exp(-jnp.abs(x)))


def _loss_body(d_ref, o_ref):
    x = d_ref[...]
    pos = x[:, 0:P]
    neg = x[:, P:ROWS]
    lp = jnp.sum(_logsig(pos), axis=1)
    ln = jnp.sum(_logsig(-neg), axis=1)
    o_ref[...] = -(lp + ln)


def _tc_loss(dots):
    blk = 1024
    return pl.pallas_call(
        _loss_body,
        grid=(BATCH // blk,),
        in_specs=[pl.BlockSpec((blk, 32), lambda i: (i, 0))],
        out_specs=pl.BlockSpec((blk,), lambda i: (i,)),
        out_shape=jax.ShapeDtypeStruct((BATCH,), jnp.float32),
    )(dots)


def kernel(input_labels, pos_labels, neg_labels, center_embedding, back_embedding):
    ctail = center_embedding[VTAIL:].reshape(-1)
    btail = back_embedding[VTAIL:].reshape(-1)
    lc, lb = _sc_linearize(center_embedding.T, back_embedding.T, ctail, btail)
    dots = _sc_dots(input_labels.astype(jnp.int32),
                    pos_labels.astype(jnp.int32).T,
                    neg_labels.astype(jnp.int32).T,
                    lc.reshape(VOCAB, EMBED),
                    lb.reshape(VOCAB, EMBED))
    return _tc_loss(dots)
